# Initial kernel scaffold; baseline (speedup 1.0000x reference)
import jax
import jax.numpy as jnp
from jax.experimental import pallas as pl


def kernel(x, edge_index, edge_weight, W, b):
    raise NotImplementedError("write your pallas kernel here")


# speedup vs baseline: 9.3942x; 9.3942x over previous
import functools

import jax
import jax.numpy as jnp
from jax import lax
from jax.experimental import pallas as pl
from jax.experimental.pallas import tpu as pltpu
from jax.experimental.pallas import tpu_sc as plsc

N_NODES = 10000
N_EDGES = 320000
D = 128

NC = 2
NS = 16
L = 16
NW = NC * NS

CH = 128
NCHUNK = 2560
E_PAD = NCHUNK * CH
CPW = NCHUNK // NW
CPT = NCHUNK // NS
BLK = 16
MBLK = CPW // BLK
DBLK = CPT // BLK

ROWB = 80
NROWCH = N_NODES // ROWB

MM_BLK = 400
MM_GRID = N_NODES // MM_BLK


def _mm_body(x_ref, w_ref, o_ref):
    o_ref[...] = jnp.dot(x_ref[...], w_ref[...], preferred_element_type=jnp.float32)


def _matmul(x, W):
    return pl.pallas_call(
        _mm_body,
        grid=(MM_GRID,),
        in_specs=[
            pl.BlockSpec((MM_BLK, D), lambda i: (i, 0)),
            pl.BlockSpec((D, D), lambda i: (0, 0)),
        ],
        out_specs=pl.BlockSpec((MM_BLK, D), lambda i: (i, 0)),
        out_shape=jax.ShapeDtypeStruct((N_NODES, D), jnp.float32),
    )(x, W)


def _fin_body(p_ref, b_ref, o_ref):
    o_ref[...] = p_ref[0] + p_ref[1] + b_ref[...]


def _final_add(parts, b):
    return pl.pallas_call(
        _fin_body,
        grid=(MM_GRID,),
        in_specs=[
            pl.BlockSpec((NC, MM_BLK, D), lambda i: (0, i, 0)),
            pl.BlockSpec((D,), lambda i: (0,)),
        ],
        out_specs=pl.BlockSpec((MM_BLK, D), lambda i: (i, 0)),
        out_shape=jax.ShapeDtypeStruct((N_NODES, D), jnp.float32),
    )(parts, b)


_MESH = plsc.VectorSubcoreMesh(core_axis_name="c", subcore_axis_name="s")


@functools.partial(
    pl.kernel,
    out_type=jax.ShapeDtypeStruct((NC, N_NODES, D), jnp.float32),
    mesh=_MESH,
    compiler_params=pltpu.CompilerParams(needs_layout_passes=False),
    scratch_types=[
        pltpu.VMEM((BLK, CH), jnp.int32),
        pltpu.VMEM((BLK, CH), jnp.int32),
        pltpu.VMEM((BLK, CH), jnp.float32),
        pltpu.VMEM((N_NODES,), jnp.float32),
        pltpu.VMEM((CH, D), jnp.float32),
        pltpu.VMEM_SHARED((N_NODES, D), jnp.float32),
        pltpu.VMEM_SHARED((N_NODES,), jnp.float32),
        pltpu.SemaphoreType.DMA,
    ],
)
def _sc_kernel(h_hbm, row_hbm, col_hbm, ew_hbm, out_hbm,
               rowb, colb, ewb, dis, msg, acc, deg, sem):
    cid = lax.axis_index("c")
    sid = lax.axis_index("s")
    wid = cid * NS + sid

    zv = jnp.zeros((L,), jnp.float32)

    def _zmsg(i, _):
        def _zc(k, _):
            msg[i, pl.ds(k * L, L)] = zv
            return 0
        return lax.fori_loop(0, D // L, _zc, 0)
    lax.fori_loop(0, CH, _zmsg, 0)

    def _zdis(i, _):
        dis[pl.ds(i * L, L)] = zv
        return 0
    lax.fori_loop(0, N_NODES // L, _zdis, 0)

    def _zacc(t, _):
        c = sid + t * NS
        @pl.when(c < NROWCH)
        def _():
            pltpu.sync_copy(msg.at[pl.ds(0, ROWB)], acc.at[pl.ds(c * ROWB, ROWB)])
        return 0
    lax.fori_loop(0, (NROWCH + NS - 1) // NS, _zacc, 0)

    @pl.when(sid == 0)
    def _():
        pltpu.sync_copy(dis, deg)

    plsc.subcore_barrier()

    def _dblk(t, _):
        dbase = sid * CPT + t * BLK
        pltpu.sync_copy(col_hbm.at[pl.ds(dbase, BLK)], colb)
        pltpu.sync_copy(ew_hbm.at[pl.ds(dbase, BLK)], ewb)

        def _dadd(j, _):
            pltpu.sync_copy(ewb.at[j], deg.at[colb.at[j]], add=True)
            return 0
        lax.fori_loop(0, BLK, _dadd, 0)
        return 0
    lax.fori_loop(0, DBLK, _dblk, 0)

    plsc.subcore_barrier()

    pltpu.sync_copy(deg, dis)
    magic = jnp.full((L,), 0x5F3759DF, jnp.int32)

    def _dloop(i, _):
        sl = pl.ds(i * L, L)
        d = dis[sl]
        y = lax.bitcast_convert_type(
            magic - (lax.bitcast_convert_type(d, jnp.int32) >> 1), jnp.float32)
        hlf = d * 0.5
        y = y * (1.5 - hlf * y * y)
        y = y * (1.5 - hlf * y * y)
        y = y * (1.5 - hlf * y * y)
        dis[sl] = jnp.where(d > 0.0, y, 0.0)
        return 0
    lax.fori_loop(0, N_NODES // L, _dloop, 0)

    def _mblk(t, _):
        wbase = wid * CPW + t * BLK
        pltpu.sync_copy(row_hbm.at[pl.ds(wbase, BLK)], rowb)
        pltpu.sync_copy(col_hbm.at[pl.ds(wbase, BLK)], colb)
        pltpu.sync_copy(ew_hbm.at[pl.ds(wbase, BLK)], ewb)

        def _nloop(j, _):
            def _kloop(k, _):
                sl = pl.ds(k * L, L)
                rv = rowb[j, sl]
                cv = colb[j, sl]
                nv = plsc.load_gather(dis, [rv]) * ewb[j, sl] * plsc.load_gather(dis, [cv])
                ewb[j, sl] = nv
                return 0
            return lax.fori_loop(0, CH // L, _kloop, 0)
        lax.fori_loop(0, BLK, _nloop, 0)

        def _mloop(j, _):
            pltpu.async_copy(h_hbm.at[rowb.at[j]], msg, sem).wait()

            def _rloop(e, _):
                jv = jnp.full((L,), j, jnp.int32)
                ev = jnp.full((L,), e, jnp.int32)
                n = plsc.load_gather(ewb, [jv, ev])
                def _cmul(k, _):
                    sl = pl.ds(k * L, L)
                    msg[e, sl] = msg[e, sl] * n
                    return 0
                return lax.fori_loop(0, D // L, _cmul, 0)
            lax.fori_loop(0, CH, _rloop, 0)

            pltpu.sync_copy(msg, acc.at[colb.at[j]], add=True)
            return 0
        lax.fori_loop(0, BLK, _mloop, 0)
        return 0
    lax.fori_loop(0, MBLK, _mblk, 0)

    plsc.subcore_barrier()

    def _oloop(t, _):
        c = sid + t * NS
        @pl.when(c < NROWCH)
        def _():
            r = c * ROWB
            pltpu.sync_copy(acc.at[pl.ds(r, ROWB)], out_hbm.at[cid, pl.ds(r, ROWB), :])
        return 0
    lax.fori_loop(0, (NROWCH + NS - 1) // NS, _oloop, 0)


def kernel(x, edge_index, edge_weight, W, b):
    pad = E_PAD - N_EDGES
    zi = jnp.zeros((pad,), jnp.int32)
    row = jnp.concatenate([edge_index[0].astype(jnp.int32), zi]).reshape(NCHUNK, CH)
    col = jnp.concatenate([edge_index[1].astype(jnp.int32), zi]).reshape(NCHUNK, CH)
    ew = jnp.concatenate([edge_weight, jnp.zeros((pad,), jnp.float32)]).reshape(NCHUNK, CH)
    h = _matmul(x, W)
    parts = _sc_kernel(h, row, col, ew)
    return _final_add(parts, b)


# speedup vs baseline: 10.7003x; 1.1390x over previous
import functools

import jax
import jax.numpy as jnp
from jax import lax
from jax.experimental import pallas as pl
from jax.experimental.pallas import tpu as pltpu
from jax.experimental.pallas import tpu_sc as plsc

N_NODES = 10000
N_EDGES = 320000
D = 128

NC = 2
NS = 16
L = 16
NW = NC * NS

CH = 128
NCHUNK = 2560
E_PAD = NCHUNK * CH
CPW = NCHUNK // NW
CPT = NCHUNK // NS
BLK = 16
MBLK = CPW // BLK
DBLK = CPT // BLK

ROWB = 80
NROWCH = N_NODES // ROWB

MM_BLK = 400
MM_GRID = N_NODES // MM_BLK


def _mm_body(x_ref, w_ref, o_ref):
    o_ref[...] = jnp.dot(x_ref[...], w_ref[...], preferred_element_type=jnp.float32)


def _matmul(x, W):
    return pl.pallas_call(
        _mm_body,
        grid=(MM_GRID,),
        in_specs=[
            pl.BlockSpec((MM_BLK, D), lambda i: (i, 0)),
            pl.BlockSpec((D, D), lambda i: (0, 0)),
        ],
        out_specs=pl.BlockSpec((MM_BLK, D), lambda i: (i, 0)),
        out_shape=jax.ShapeDtypeStruct((N_NODES, D), jnp.float32),
    )(x, W)


def _fin_body(p_ref, b_ref, o_ref):
    o_ref[...] = p_ref[0] + p_ref[1] + b_ref[...]


def _final_add(parts, b):
    return pl.pallas_call(
        _fin_body,
        grid=(MM_GRID,),
        in_specs=[
            pl.BlockSpec((NC, MM_BLK, D), lambda i: (0, i, 0)),
            pl.BlockSpec((D,), lambda i: (0,)),
        ],
        out_specs=pl.BlockSpec((MM_BLK, D), lambda i: (i, 0)),
        out_shape=jax.ShapeDtypeStruct((N_NODES, D), jnp.float32),
    )(parts, b)


_MESH = plsc.VectorSubcoreMesh(core_axis_name="c", subcore_axis_name="s")


@functools.partial(
    pl.kernel,
    out_type=jax.ShapeDtypeStruct((NC, N_NODES, D), jnp.float32),
    mesh=_MESH,
    compiler_params=pltpu.CompilerParams(needs_layout_passes=False),
    scratch_types=[
        pltpu.VMEM((BLK, CH), jnp.int32),
        pltpu.VMEM((BLK, CH), jnp.int32),
        pltpu.VMEM((BLK, CH), jnp.float32),
        pltpu.VMEM((N_NODES,), jnp.float32),
        pltpu.VMEM((CH, D), jnp.float32),
        pltpu.VMEM((CH, D), jnp.float32),
        pltpu.VMEM_SHARED((N_NODES, D), jnp.float32),
        pltpu.VMEM_SHARED((N_NODES,), jnp.float32),
        pltpu.SemaphoreType.DMA,
        pltpu.SemaphoreType.DMA,
        pltpu.SemaphoreType.DMA,
        pltpu.SemaphoreType.DMA,
        pltpu.SemaphoreType.DMA,
    ],
)
def _sc_kernel(h_hbm, row_hbm, col_hbm, ew_hbm, out_hbm,
               rowb, colb, ewb, dis, msgA, msgB, acc, deg,
               gsA, gsB, ssA, ssB, dsem):
    cid = lax.axis_index("c")
    sid = lax.axis_index("s")
    wid = cid * NS + sid

    zv = jnp.zeros((L,), jnp.float32)

    def _zmsg(i, _):
        def _zc(k, _):
            msgA[i, pl.ds(k * L, L)] = zv
            return 0
        return lax.fori_loop(0, D // L, _zc, 0)
    lax.fori_loop(0, CH, _zmsg, 0)

    def _zdis(i, _):
        dis[pl.ds(i * L, L)] = zv
        return 0
    lax.fori_loop(0, N_NODES // L, _zdis, 0)

    def _zacc(t, _):
        c = sid + t * NS
        @pl.when(c < NROWCH)
        def _():
            pltpu.sync_copy(msgA.at[pl.ds(0, ROWB)], acc.at[pl.ds(c * ROWB, ROWB)])
        return 0
    lax.fori_loop(0, (NROWCH + NS - 1) // NS, _zacc, 0)

    @pl.when(sid == 0)
    def _():
        pltpu.sync_copy(dis, deg)

    plsc.subcore_barrier()

    def _dblk(t, _):
        dbase = sid * CPT + t * BLK
        pltpu.sync_copy(col_hbm.at[pl.ds(dbase, BLK)], colb)
        pltpu.sync_copy(ew_hbm.at[pl.ds(dbase, BLK)], ewb)

        def _dadd(j, _):
            pltpu.async_copy(ewb.at[j], deg.at[colb.at[j]], dsem, add=True)
            return 0
        lax.fori_loop(0, BLK, _dadd, 0)

        def _ddrain(j, _):
            pltpu.make_async_copy(ewb.at[j], deg.at[colb.at[j]], dsem).wait()
            return 0
        lax.fori_loop(0, BLK, _ddrain, 0)
        return 0
    lax.fori_loop(0, DBLK, _dblk, 0)

    plsc.subcore_barrier()

    pltpu.sync_copy(deg, dis)
    magic = jnp.full((L,), 0x5F3759DF, jnp.int32)

    def _dloop(i, _):
        sl = pl.ds(i * L, L)
        d = dis[sl]
        y = lax.bitcast_convert_type(
            magic - (lax.bitcast_convert_type(d, jnp.int32) >> 1), jnp.float32)
        hlf = d * 0.5
        y = y * (1.5 - hlf * y * y)
        y = y * (1.5 - hlf * y * y)
        y = y * (1.5 - hlf * y * y)
        dis[sl] = jnp.where(d > 0.0, y, 0.0)
        return 0
    lax.fori_loop(0, N_NODES // L, _dloop, 0)

    def _mblk(t, _):
        wbase = wid * CPW + t * BLK
        pltpu.sync_copy(row_hbm.at[pl.ds(wbase, BLK)], rowb)
        pltpu.sync_copy(col_hbm.at[pl.ds(wbase, BLK)], colb)
        pltpu.sync_copy(ew_hbm.at[pl.ds(wbase, BLK)], ewb)

        def _nloop(j, _):
            def _kloop(k, _):
                sl = pl.ds(k * L, L)
                rv = rowb[j, sl]
                cv = colb[j, sl]
                nv = plsc.load_gather(dis, [rv]) * ewb[j, sl] * plsc.load_gather(dis, [cv])
                ewb[j, sl] = nv
                return 0
            return lax.fori_loop(0, CH // L, _kloop, 0)
        lax.fori_loop(0, BLK, _nloop, 0)

        def _scale(mref, j):
            def _rloop(e, _):
                jv = jnp.full((L,), j, jnp.int32)
                ev = jnp.full((L,), e, jnp.int32)
                n = plsc.load_gather(ewb, [jv, ev])
                for k in range(D // L):
                    sl = pl.ds(k * L, L)
                    mref[e, sl] = mref[e, sl] * n
                return 0
            lax.fori_loop(0, CH, _rloop, 0)

        pltpu.async_copy(h_hbm.at[rowb.at[0]], msgA, gsA)

        def _mpair(p, _):
            jA = 2 * p
            jB = 2 * p + 1

            @pl.when(p > 0)
            def _():
                pltpu.make_async_copy(msgB, acc.at[colb.at[jB - 2]], ssB).wait()
            pltpu.async_copy(h_hbm.at[rowb.at[jB]], msgB, gsB)

            pltpu.make_async_copy(h_hbm.at[rowb.at[jA]], msgA, gsA).wait()
            _scale(msgA, jA)
            pltpu.async_copy(msgA, acc.at[colb.at[jA]], ssA, add=True)

            pltpu.make_async_copy(h_hbm.at[rowb.at[jB]], msgB, gsB).wait()
            _scale(msgB, jB)

            @pl.when(p < BLK // 2 - 1)
            def _():
                pltpu.make_async_copy(msgA, acc.at[colb.at[jA]], ssA).wait()
                pltpu.async_copy(h_hbm.at[rowb.at[jA + 2]], msgA, gsA)

            pltpu.async_copy(msgB, acc.at[colb.at[jB]], ssB, add=True)
            return 0
        lax.fori_loop(0, BLK // 2, _mpair, 0)

        pltpu.make_async_copy(msgA, acc.at[colb.at[BLK - 2]], ssA).wait()
        pltpu.make_async_copy(msgB, acc.at[colb.at[BLK - 1]], ssB).wait()
        return 0
    lax.fori_loop(0, MBLK, _mblk, 0)

    plsc.subcore_barrier()

    def _oloop(t, _):
        c = sid + t * NS
        @pl.when(c < NROWCH)
        def _():
            r = c * ROWB
            pltpu.sync_copy(acc.at[pl.ds(r, ROWB)], out_hbm.at[cid, pl.ds(r, ROWB), :])
        return 0
    lax.fori_loop(0, (NROWCH + NS - 1) // NS, _oloop, 0)


def kernel(x, edge_index, edge_weight, W, b):
    pad = E_PAD - N_EDGES
    zi = jnp.zeros((pad,), jnp.int32)
    row = jnp.concatenate([edge_index[0].astype(jnp.int32), zi]).reshape(NCHUNK, CH)
    col = jnp.concatenate([edge_index[1].astype(jnp.int32), zi]).reshape(NCHUNK, CH)
    ew = jnp.concatenate([edge_weight, jnp.zeros((pad,), jnp.float32)]).reshape(NCHUNK, CH)
    h = _matmul(x, W)
    parts = _sc_kernel(h, row, col, ew)
    return _final_add(parts, b)
